# baseline (speedup 1.0000x reference)
import numpy as np
import jax
import jax.numpy as jnp
from jax import lax
from jax.experimental import pallas as pl

RES = 8
NOISE_STD = 0.1
NUM_STROKES = 12
NUM_VARIANTS = RES * RES
NUM_COLS = NUM_STROKES * NUM_VARIANTS
NUM_LATENTS = NUM_STROKES * NUM_STROKES


def _build_images_np():
    rotations = [(0, 1), (1, 0), (1, 1), (1, -1)]
    lengths = list(range(2, RES // 2 + 1))
    stroke_types = [(l, r) for l in lengths for r in rotations]
    images_per_stroke = []
    for length, (right, up) in stroke_types:
        i_off = np.arange(length) * right
        j_off = np.arange(length) * up
        variants = []
        for i in range(RES):
            for j in range(RES):
                i_s = i + i_off
                j_s = j + j_off
                ok = (i_s >= 0) & (i_s < RES) & (j_s >= 0) & (j_s < RES)
                adapted_len = int(ok.sum())
                img = np.zeros((RES, RES), dtype=np.float32)
                img[i_s[:adapted_len], j_s[:adapted_len]] = 1.0
                variants.append(img)
        images_per_stroke.append(np.stack(variants))
    return np.stack(images_per_stroke)


_IMG = _build_images_np()
_WT = _IMG.reshape(NUM_COLS, RES * RES)
_ISQ = _WT.sum(axis=1)

_LOG2E = np.float64(1.0 / np.log(2.0))
_SCALE = np.float64(1.0 / (2.0 * NOISE_STD * NOISE_STD))
_CONST = np.float64(
    RES * RES * (-np.log(NOISE_STD) - 0.5 * np.log(2.0 * np.pi))
    - np.log(NUM_VARIANTS)
)

_KDIM = 2 * RES * RES + 8
_M = np.zeros((NUM_COLS + 8, _KDIM), dtype=np.float32)
_M[:NUM_COLS, : RES * RES] = _WT * np.float32(2.0 * _SCALE * _LOG2E)
_M[:NUM_COLS, 2 * RES * RES] = -(_ISQ * _SCALE * _LOG2E).astype(np.float32)
_M[NUM_COLS, RES * RES : 2 * RES * RES] = np.float32(_SCALE * _LOG2E)

_K = np.arange(NUM_LATENTS)
_E1 = (np.equal.outer(np.arange(NUM_STROKES), _K // NUM_STROKES)).astype(np.float32)
_E2 = (np.equal.outer(np.arange(NUM_STROKES), _K % NUM_STROKES)).astype(np.float32)
_EXP = np.concatenate([_E1, _E2 - _E1], axis=0)

_LN2 = np.float32(np.log(2.0))
_CONST_B2 = np.float32(_CONST * _LOG2E)

B_BLK = 8192


def _stroke_kernel(x_ref, y_ref, m_ref, exp_ref, out_ref):
    xf = x_ref[...]
    ones8 = jnp.ones((xf.shape[0], 8), jnp.float32)
    a = jnp.concatenate([xf, xf * xf, ones8], axis=1)
    t = lax.dot_general(
        m_ref[...], a, (((1,), (1,)), ((), ())), preferred_element_type=jnp.float32
    )
    xsqh = t[NUM_COLS : NUM_COLS + 1]
    z3 = t[:NUM_COLS].reshape(NUM_STROKES, NUM_VARIANTS, t.shape[1])
    m = jnp.max(z3, axis=1)
    e3 = jnp.exp2(z3 - m[:, None, :])
    s = jnp.sum(e3, axis=1)
    ll12t = (m + jnp.log2(s) + (_CONST_B2 - xsqh)) * _LN2
    pad = jnp.zeros((4, ll12t.shape[1]), jnp.float32)
    ll12 = jnp.concatenate([ll12t, pad], axis=0).T[:, :NUM_STROKES]
    ybf = (y_ref[...] != 0).astype(jnp.float32)
    lhs = jnp.concatenate([ll12, ll12 * ybf], axis=1)
    out_ref[...] = jnp.dot(lhs, exp_ref[...], preferred_element_type=jnp.float32)


def kernel(x, y):
    B = x.shape[0]
    xf = x.reshape(B, RES * RES)
    y2 = y.reshape(B, 1)
    mm = jnp.asarray(_M)
    expm = jnp.asarray(_EXP)
    grid = (B // B_BLK,)
    out = pl.pallas_call(
        _stroke_kernel,
        grid=grid,
        in_specs=[
            pl.BlockSpec((B_BLK, RES * RES), lambda i: (i, 0)),
            pl.BlockSpec((B_BLK, 1), lambda i: (i, 0)),
            pl.BlockSpec(_M.shape, lambda i: (0, 0)),
            pl.BlockSpec(_EXP.shape, lambda i: (0, 0)),
        ],
        out_specs=pl.BlockSpec((B_BLK, NUM_LATENTS), lambda i: (i, 0)),
        out_shape=jax.ShapeDtypeStruct((B, NUM_LATENTS), jnp.float32),
    )(xf, y2, mm, expm)
    return out


# speedup vs baseline: 1.0502x; 1.0502x over previous
import numpy as np
import jax
import jax.numpy as jnp
from jax import lax
from jax.experimental import pallas as pl

RES = 8
NOISE_STD = 0.1
NUM_STROKES = 12
NUM_VARIANTS = RES * RES
NUM_COLS = NUM_STROKES * NUM_VARIANTS
NUM_LATENTS = NUM_STROKES * NUM_STROKES


def _build_images_np():
    rotations = [(0, 1), (1, 0), (1, 1), (1, -1)]
    lengths = list(range(2, RES // 2 + 1))
    stroke_types = [(l, r) for l in lengths for r in rotations]
    images_per_stroke = []
    for length, (right, up) in stroke_types:
        i_off = np.arange(length) * right
        j_off = np.arange(length) * up
        variants = []
        for i in range(RES):
            for j in range(RES):
                i_s = i + i_off
                j_s = j + j_off
                ok = (i_s >= 0) & (i_s < RES) & (j_s >= 0) & (j_s < RES)
                adapted_len = int(ok.sum())
                img = np.zeros((RES, RES), dtype=np.float32)
                img[i_s[:adapted_len], j_s[:adapted_len]] = 1.0
                variants.append(img)
        images_per_stroke.append(np.stack(variants))
    return np.stack(images_per_stroke)


_IMG = _build_images_np()
_WT = _IMG.reshape(NUM_COLS, RES * RES)
_ISQ = _WT.sum(axis=1)

_LOG2E = np.float64(1.0 / np.log(2.0))
_SCALE = np.float64(1.0 / (2.0 * NOISE_STD * NOISE_STD))
_CONST = np.float64(
    RES * RES * (-np.log(NOISE_STD) - 0.5 * np.log(2.0 * np.pi))
    - np.log(NUM_VARIANTS)
)

_KDIM = 2 * RES * RES + 8
_M = np.zeros((NUM_COLS + 8, _KDIM), dtype=np.float32)
_M[:NUM_COLS, : RES * RES] = _WT * np.float32(2.0 * _SCALE * _LOG2E)
_M[:NUM_COLS, 2 * RES * RES] = -(_ISQ * _SCALE * _LOG2E).astype(np.float32)
_M[NUM_COLS, RES * RES : 2 * RES * RES] = np.float32(_SCALE * _LOG2E)

_K = np.arange(NUM_LATENTS)
_E1 = (np.equal.outer(np.arange(NUM_STROKES), _K // NUM_STROKES)).astype(np.float32)
_E2 = (np.equal.outer(np.arange(NUM_STROKES), _K % NUM_STROKES)).astype(np.float32)
_EXP = np.concatenate([_E1, _E2 - _E1], axis=0)

_LN2 = np.float32(np.log(2.0))
_CONST_B2 = np.float32(_CONST * _LOG2E)

B_BLK = 4096


def _stroke_kernel(x_ref, y_ref, m_ref, exp_ref, out_ref):
    xf = x_ref[...]
    ones8 = jnp.ones((xf.shape[0], 8), jnp.float32)
    a = jnp.concatenate([xf, xf * xf, ones8], axis=1)
    t = lax.dot_general(
        m_ref[...], a, (((1,), (1,)), ((), ())), preferred_element_type=jnp.float32
    )
    xsqh = t[NUM_COLS : NUM_COLS + 1]
    z3 = t[:NUM_COLS].reshape(NUM_STROKES, NUM_VARIANTS, t.shape[1])
    m = jnp.max(z3, axis=1)
    e3 = jnp.exp2(z3 - m[:, None, :])
    s = jnp.sum(e3, axis=1)
    ll12t = (m + jnp.log2(s) + (_CONST_B2 - xsqh)) * _LN2
    pad = jnp.zeros((4, ll12t.shape[1]), jnp.float32)
    ll12 = jnp.concatenate([ll12t, pad], axis=0).T[:, :NUM_STROKES]
    ybf = (y_ref[...] != 0).astype(jnp.float32)
    lhs = jnp.concatenate([ll12, ll12 * ybf], axis=1)
    out_ref[...] = jnp.dot(lhs, exp_ref[...], preferred_element_type=jnp.float32)


def kernel(x, y):
    B = x.shape[0]
    xf = x.reshape(B, RES * RES)
    y2 = y.reshape(B, 1)
    mm = jnp.asarray(_M)
    expm = jnp.asarray(_EXP)
    grid = (B // B_BLK,)
    out = pl.pallas_call(
        _stroke_kernel,
        grid=grid,
        in_specs=[
            pl.BlockSpec((B_BLK, RES * RES), lambda i: (i, 0)),
            pl.BlockSpec((B_BLK, 1), lambda i: (i, 0)),
            pl.BlockSpec(_M.shape, lambda i: (0, 0)),
            pl.BlockSpec(_EXP.shape, lambda i: (0, 0)),
        ],
        out_specs=pl.BlockSpec((B_BLK, NUM_LATENTS), lambda i: (i, 0)),
        out_shape=jax.ShapeDtypeStruct((B, NUM_LATENTS), jnp.float32),
    )(xf, y2, mm, expm)
    return out
